# baseline (speedup 1.0000x reference)
import numpy as np
import jax
import jax.numpy as jnp
from jax.experimental import pallas as pl
from jax.experimental.pallas import tpu as pltpu

_TAU = 0.1
_NUM_NEG = 64
_PROX = 40
_C, _H, _W = 128, 84, 84
_NOFF = 4
_NSHIFT = _NOFF * _NOFF
_HR = _H + _NOFF - 1
_NSPLIT = 4
_HB = _H // _NSPLIT


def _neg_shift_counts() -> np.ndarray:
    n = _H * _W
    rng = np.random.default_rng(0)
    off_r = rng.integers(_PROX, _H - _PROX, size=(n, _NUM_NEG))
    off_c = rng.integers(_PROX, _W - _PROX, size=(n, _NUM_NEG))
    s = (off_r - _PROX) * _NOFF + (off_c - _PROX)
    cnt = np.zeros((n, _NSHIFT), np.uint8)
    np.add.at(cnt, (np.arange(n)[:, None], s), 1)
    return np.ascontiguousarray(
        cnt.reshape(_H, _W, _NSHIFT).transpose(0, 2, 1))


_CNT = _neg_shift_counts()


def _loss_kernel(emb_ref, mom_ref, cnt_ref, mask_ref, out_ref,
                 col_ref, t1_ref, sl_ref, sm_ref):
    inv_tau = 1.0 / _TAU
    pid = pl.program_id(0)

    @pl.when(pid == 0)
    def _build():
        t1_ref[...] = jnp.zeros_like(t1_ref)
        sl_ref[...] = jnp.zeros_like(sl_ref)
        sm_ref[...] = jnp.zeros_like(sm_ref)
        top = _H - _PROX
        for j in range(_NOFF):
            wsplit = _H - _PROX - j
            col_ref[j, 0:top, 0:wsplit] = mom_ref[_PROX:_H, _PROX + j:_H]
            col_ref[j, 0:top, wsplit:_W] = mom_ref[_PROX:_H, 0:_PROX + j]
            col_ref[j, top:_HR, 0:wsplit] = mom_ref[0:_HR - top,
                                                    _PROX + j:_H]
            col_ref[j, top:_HR, wsplit:_W] = mom_ref[0:_HR - top,
                                                     0:_PROX + j]

    h0 = pid * _HB
    emb = emb_ref[...]
    pos = jnp.sum(emb * mom_ref[pl.ds(h0, _HB)], axis=-1)
    dpos = (1.0 + pos) * inv_tau

    s_sum = jnp.exp(dpos)
    for s in range(_NSHIFT):
        dr = s // _NOFF
        dc = s % _NOFF
        mom_s = col_ref[dc, pl.ds(h0 + dr, _HB)]
        d = jnp.sum(emb * mom_s, axis=-1)
        cnt_s = cnt_ref[:, s, :].astype(jnp.float32)
        s_sum = s_sum + cnt_s * jnp.exp((1.0 + d) * inv_tau)

    m = mask_ref[0]
    t1_ref[...] += jnp.sum(m * (-dpos))[None, None]
    sl_ref[...] += jnp.sum(jnp.log(s_sum))[None, None]
    sm_ref[...] += jnp.sum(m)[None, None]

    @pl.when(pid == _NSPLIT - 1)
    def _finish():
        n = float(_H * _W)
        out_ref[...] = (n * t1_ref[...] + sl_ref[...] * sm_ref[...]) / (n * n)


def kernel(embeddings, mom_embeddings, k, mask, warmup):
    emb = jnp.transpose(embeddings, (1, 2, 0))
    mom = jnp.transpose(mom_embeddings, (1, 2, 0))
    out = pl.pallas_call(
        _loss_kernel,
        grid=(_NSPLIT,),
        in_specs=[
            pl.BlockSpec((_HB, _W, _C), lambda i: (i, 0, 0)),
            pl.BlockSpec((_H, _W, _C), lambda i: (0, 0, 0)),
            pl.BlockSpec((_HB, _NSHIFT, _W), lambda i: (i, 0, 0)),
            pl.BlockSpec((1, _HB, _W), lambda i: (i, 0, 0)),
        ],
        out_specs=pl.BlockSpec((1, 1), lambda i: (0, 0)),
        out_shape=jax.ShapeDtypeStruct((1, 1), jnp.float32),
        scratch_shapes=[pltpu.VMEM((_NOFF, _HR, _W, _C), jnp.float32),
                        pltpu.VMEM((1, 1), jnp.float32),
                        pltpu.VMEM((1, 1), jnp.float32),
                        pltpu.VMEM((1, 1), jnp.float32)],
    )(emb, mom, jnp.asarray(_CNT), mask.reshape(_NSPLIT, _HB, _W))
    return out[0, 0]


# speedup vs baseline: 1.5089x; 1.5089x over previous
import numpy as np
import jax
import jax.numpy as jnp
from jax.experimental import pallas as pl
from jax.experimental.pallas import tpu as pltpu

_TAU = 0.1
_NUM_NEG = 64
_PROX = 40
_C, _H, _W = 128, 84, 84
_NOFF = 4
_NSHIFT = _NOFF * _NOFF
_HR = _H + _NOFF - 1
_NCHUNK = 8
_CB = _C // _NCHUNK


def _neg_shift_counts() -> np.ndarray:
    n = _H * _W
    rng = np.random.default_rng(0)
    off_r = rng.integers(_PROX, _H - _PROX, size=(n, _NUM_NEG))
    off_c = rng.integers(_PROX, _W - _PROX, size=(n, _NUM_NEG))
    s = (off_r - _PROX) * _NOFF + (off_c - _PROX)
    cnt = np.zeros((n, _NSHIFT), np.uint8)
    np.add.at(cnt, (np.arange(n)[:, None], s), 1)
    return np.ascontiguousarray(cnt.T).reshape(_NSHIFT, _H, _W)


_CNT = _neg_shift_counts()


def _loss_kernel(emb_ref, mom_ref, cnt_ref, mask_ref, out_ref,
                 col_ref, d_ref):
    inv_tau = 1.0 / _TAU
    pid = pl.program_id(0)

    @pl.when(pid == 0)
    def _init():
        d_ref[...] = jnp.zeros_like(d_ref)

    for j in range(_NOFF):
        top = _H - _PROX
        wsplit = _H - _PROX - j
        col_ref[j, :, 0:top, 0:wsplit] = mom_ref[:, _PROX:_H, _PROX + j:_H]
        col_ref[j, :, 0:top, wsplit:_W] = mom_ref[:, _PROX:_H, 0:_PROX + j]
        col_ref[j, :, top:_HR, 0:wsplit] = mom_ref[:, 0:_HR - top,
                                                   _PROX + j:_H]
        col_ref[j, :, top:_HR, wsplit:_W] = mom_ref[:, 0:_HR - top,
                                                    0:_PROX + j]

    d_ref[0] += jnp.sum(emb_ref[...] * mom_ref[...], axis=0)
    for s in range(_NSHIFT):
        dr = s // _NOFF
        dc = s % _NOFF
        mom_s = col_ref[dc, :, dr:dr + _H, :]
        d_ref[1 + s] += jnp.sum(emb_ref[...] * mom_s, axis=0)

    @pl.when(pid == _NCHUNK - 1)
    def _finish():
        dpos = (1.0 + d_ref[0]) * inv_tau
        s_sum = jnp.exp(dpos)
        for s in range(_NSHIFT):
            cnt_s = cnt_ref[s].astype(jnp.float32)
            s_sum = s_sum + cnt_s * jnp.exp((1.0 + d_ref[1 + s]) * inv_tau)
        m = mask_ref[...]
        n = float(_H * _W)
        loss = (n * jnp.sum(m * (-dpos))
                + jnp.sum(jnp.log(s_sum)) * jnp.sum(m)) / (n * n)
        out_ref[...] = loss[None, None]


def kernel(embeddings, mom_embeddings, k, mask, warmup):
    out = pl.pallas_call(
        _loss_kernel,
        grid=(_NCHUNK,),
        in_specs=[
            pl.BlockSpec((_CB, _H, _W), lambda i: (i, 0, 0)),
            pl.BlockSpec((_CB, _H, _W), lambda i: (i, 0, 0)),
            pl.BlockSpec((_NSHIFT, _H, _W), lambda i: (0, 0, 0)),
            pl.BlockSpec((_H, _W), lambda i: (0, 0)),
        ],
        out_specs=pl.BlockSpec((1, 1), lambda i: (0, 0)),
        out_shape=jax.ShapeDtypeStruct((1, 1), jnp.float32),
        scratch_shapes=[pltpu.VMEM((_NOFF, _CB, _HR, _W), jnp.float32),
                        pltpu.VMEM((1 + _NSHIFT, _H, _W), jnp.float32)],
    )(embeddings, mom_embeddings, jnp.asarray(_CNT), mask)
    return out[0, 0]
